# Initial kernel scaffold; baseline (speedup 1.0000x reference)
import jax
import jax.numpy as jnp
from jax.experimental import pallas as pl


def kernel(states_batch, slab):
    raise NotImplementedError("write your pallas kernel here")


# speedup vs baseline: 1.2889x; 1.2889x over previous
import jax
import jax.numpy as jnp
from jax.experimental import pallas as pl
from jax.experimental.pallas import tpu as pltpu

_NS = 15
_AUG = 33
_LANES = 128
_BLOCK_B = 8192


def _mlp_kernel(x_ref, w1_ref, w2_ref, w3_ref, o_ref):
    x = x_ref[...]
    w1 = w1_ref[0:_NS, :]
    b1 = w1_ref[_NS:_NS + 1, :]
    h1 = jnp.maximum(
        jnp.dot(x, w1, preferred_element_type=jnp.float32) + b1, 0.0)
    h2 = jnp.maximum(
        jnp.dot(h1, w2_ref[...], preferred_element_type=jnp.float32), 0.0)
    out = jax.lax.dot_general(
        w3_ref[0:1, :], h2, (((1,), (1,)), ((), ())),
        preferred_element_type=jnp.float32)
    o_ref[...] = out[None]


def _prep_weights(slab):
    slab = jnp.asarray(slab, jnp.float32)
    a1 = slab[0:_AUG, 0:16].T
    w1p = jnp.zeros((16, _LANES), jnp.float32).at[:, 0:_AUG].set(a1)
    a2 = slab[128:128 + _AUG, 0:_AUG].T
    w2p = jnp.zeros((_LANES, _LANES), jnp.float32).at[0:_AUG, 0:_AUG].set(a2)
    w3p = jnp.zeros((8, _LANES), jnp.float32).at[0, 0:_AUG].set(slab[256, 0:_AUG])
    return w1p, w2p, w3p


def kernel(states_batch, slab):
    x = jnp.asarray(states_batch, jnp.float32)
    b, n = x.shape
    w1p, w2p, w3p = _prep_weights(slab)

    bb = _BLOCK_B
    nb = pl.cdiv(b, bb)
    out = pl.pallas_call(
        _mlp_kernel,
        out_shape=jax.ShapeDtypeStruct((nb, 1, bb), jnp.float32),
        grid=(nb,),
        in_specs=[
            pl.BlockSpec((bb, n), lambda i: (i, 0)),
            pl.BlockSpec((16, _LANES), lambda i: (0, 0)),
            pl.BlockSpec((_LANES, _LANES), lambda i: (0, 0)),
            pl.BlockSpec((8, _LANES), lambda i: (0, 0)),
        ],
        out_specs=pl.BlockSpec((1, 1, bb), lambda i: (i, 0, 0)),
        compiler_params=pltpu.CompilerParams(
            dimension_semantics=("parallel",)),
    )(x, w1p, w2p, w3p)
    return out.reshape(-1)[:b]


# speedup vs baseline: 1.3136x; 1.0192x over previous
import jax
import jax.numpy as jnp
from jax.experimental import pallas as pl
from jax.experimental.pallas import tpu as pltpu

_NS = 15
_K2 = 40
_BLOCK_B = 8192

_TB = (((1,), (1,)), ((), ()))


def _mlp_kernel(x_ref, slab_ref, o_ref):
    x = x_ref[...]
    s1 = slab_ref[0:_K2, 0:_NS]
    b1 = jnp.transpose(slab_ref[0:_K2, _NS:_NS + 1])
    s2 = slab_ref[128:128 + _K2, 0:_K2]
    s3 = slab_ref[256:257, 0:_K2]

    h1 = jnp.maximum(
        jax.lax.dot_general(x, s1, _TB, preferred_element_type=jnp.float32)
        + b1, 0.0)
    h2 = jnp.maximum(
        jax.lax.dot_general(h1, s2, _TB, preferred_element_type=jnp.float32),
        0.0)
    out = jax.lax.dot_general(s3, h2, _TB,
                              preferred_element_type=jnp.float32)
    o_ref[...] = out[None]


def kernel(states_batch, slab):
    x = states_batch
    b, n = x.shape
    bb = _BLOCK_B
    nb = pl.cdiv(b, bb)
    out = pl.pallas_call(
        _mlp_kernel,
        out_shape=jax.ShapeDtypeStruct((nb, 1, bb), jnp.float32),
        grid=(nb,),
        in_specs=[
            pl.BlockSpec((bb, n), lambda i: (i, 0)),
            pl.BlockSpec((264, 128), lambda i: (0, 0)),
        ],
        out_specs=pl.BlockSpec((1, 1, bb), lambda i: (i, 0, 0)),
        compiler_params=pltpu.CompilerParams(
            dimension_semantics=("parallel",)),
    )(x, slab)
    return out.reshape(-1)[:b]


# speedup vs baseline: 8.9162x; 6.7876x over previous
import jax
import jax.numpy as jnp
from jax.experimental import pallas as pl
from jax.experimental.pallas import tpu as pltpu

_NS = 15
_K2 = 40
_BLOCK_N = 16384


def _mlp_kernel(x_ref, slab_ref, o_ref):
    x = x_ref[...]
    ones = jnp.ones((1, x.shape[1]), jnp.float32)
    xa = jnp.concatenate([x, ones], axis=0)
    s1 = slab_ref[0:_K2, 0:16]
    s2 = slab_ref[128:128 + _K2, 0:_K2]
    s3 = slab_ref[256:257, 0:_K2]

    h1 = jnp.maximum(
        jnp.dot(s1, xa, preferred_element_type=jnp.float32), 0.0)
    h2 = jnp.maximum(
        jnp.dot(s2, h1, preferred_element_type=jnp.float32), 0.0)
    out = jnp.dot(s3, h2, preferred_element_type=jnp.float32)
    o_ref[...] = out[None]


def kernel(states_batch, slab):
    b, n = states_batch.shape
    x_t = jnp.transpose(states_batch)
    bn = _BLOCK_N
    nb = pl.cdiv(b, bn)
    out = pl.pallas_call(
        _mlp_kernel,
        out_shape=jax.ShapeDtypeStruct((nb, 1, bn), jnp.float32),
        grid=(nb,),
        in_specs=[
            pl.BlockSpec((n, bn), lambda i: (0, i)),
            pl.BlockSpec((264, 128), lambda i: (0, 0)),
        ],
        out_specs=pl.BlockSpec((1, 1, bn), lambda i: (i, 0, 0)),
        compiler_params=pltpu.CompilerParams(
            dimension_semantics=("parallel",)),
    )(x_t, slab)
    return out.reshape(-1)[:b]


# speedup vs baseline: 10.6603x; 1.1956x over previous
import jax
import jax.numpy as jnp
from jax.experimental import pallas as pl
from jax.experimental.pallas import tpu as pltpu

_NS = 15
_K2 = 40
_BLOCK_N = 32768


def _mlp_kernel(x_ref, slab_ref, o_ref):
    x = x_ref[...]
    ones = jnp.ones((1, x.shape[1]), jnp.float32)
    xa = jnp.concatenate([x, ones], axis=0)
    s1 = slab_ref[0:_K2, 0:16]
    s2 = slab_ref[128:128 + _K2, 0:_K2]
    s3 = slab_ref[256:257, 0:_K2]

    h1 = jnp.maximum(
        jnp.dot(s1, xa, preferred_element_type=jnp.float32), 0.0)
    h2 = jnp.maximum(
        jnp.dot(s2, h1, preferred_element_type=jnp.float32), 0.0)
    out = jnp.dot(s3, h2, preferred_element_type=jnp.float32)
    o_ref[...] = out[None]


def kernel(states_batch, slab):
    b, n = states_batch.shape
    x_t = jnp.transpose(states_batch)
    bn = _BLOCK_N
    nb = pl.cdiv(b, bn)
    out = pl.pallas_call(
        _mlp_kernel,
        out_shape=jax.ShapeDtypeStruct((nb, 1, bn), jnp.float32),
        grid=(nb,),
        in_specs=[
            pl.BlockSpec((n, bn), lambda i: (0, i)),
            pl.BlockSpec((264, 128), lambda i: (0, 0)),
        ],
        out_specs=pl.BlockSpec((1, 1, bn), lambda i: (i, 0, 0)),
        compiler_params=pltpu.CompilerParams(
            dimension_semantics=("parallel",)),
    )(x_t, slab)
    return out.reshape(-1)[:b]
